# baseline (speedup 1.0000x reference)
import functools
import math

import jax
import jax.numpy as jnp
import numpy as np
from jax import lax
from jax.experimental import pallas as pl
from jax.experimental.pallas import tpu as pltpu
from jax.experimental.pallas import tpu_sc as plsc

_VOCAB = 1000000
_D = 32
_DP = 33
_SCALE = np.float32(math.sqrt(_D))
_NC = 2
_NS = 16
_NW = _NC * _NS

_CB = 512
_NMB = _VOCAB // _CB
_TAIL = _VOCAB - _NMB * _CB
_BPW = _NMB // _NW
_EXTRA = _NMB - _BPW * _NW

_B = 4096
_S = 200
_UB = 512
_NQ = _B // _UB
_UPW = _S * _NQ // _NW
_RPAD = 40


def _mesh():
    return plsc.VectorSubcoreMesh(core_axis_name="c", subcore_axis_name="s")


@functools.partial(
    pl.kernel,
    mesh=_mesh(),
    out_type=jax.ShapeDtypeStruct((_VOCAB * _DP,), jnp.float32),
    scratch_types=[
        pltpu.VMEM((_D, _CB), jnp.float32),
        pltpu.VMEM((_D, _CB), jnp.float32),
        pltpu.VMEM((_CB * _DP,), jnp.float32),
        pltpu.VMEM((_CB * _DP,), jnp.float32),
        [pltpu.SemaphoreType.DMA] * 2,
        [pltpu.SemaphoreType.DMA] * 2,
    ],
    compiler_params=pltpu.CompilerParams(use_tc_tiling_on_sc=True, needs_layout_passes=False),
)
def _detile(lutT_hbm, tail_hbm, out_hbm, tbuf0, tbuf1, obuf0, obuf1, isems, osems):
    tbufs = (tbuf0, tbuf1)
    obufs = (obuf0, obuf1)
    wid = lax.axis_index("s") * _NC + lax.axis_index("c")
    lo = wid * _BPW + jnp.minimum(wid, _EXTRA)

    lane = lax.iota(jnp.int32, 16)
    ivec = lane * _DP

    def load(i, p):
        return pltpu.async_copy(
            lutT_hbm.at[:, pl.ds((lo + i) * _CB, _CB)], tbufs[p], isems[p]
        )

    def transpose(p, ncol):
        @plsc.parallel_loop(0, ncol // 16, 1, unroll=4)
        def _(c0):
            icvec = ivec + c0 * (16 * _DP)
            for dd in range(_D):
                val = tbufs[p][dd, pl.ds(c0 * 16, 16)]
                plsc.store_scatter(obufs[p], [icvec + dd], val)

    def store(i, p, nrow):
        return pltpu.async_copy(
            obufs[p].at[pl.ds(0, nrow * _DP)],
            out_hbm.at[pl.ds((lo + i) * (_CB * _DP), nrow * _DP)],
            osems[p],
        )

    load(0, 0)

    def pair_body(i2, c):
        for k in (0, 1):
            i = 2 * i2 + k
            p = k

            @pl.when(i + 1 < _BPW)
            def _():
                load(i + 1, 1 - p)

            pltpu.make_async_copy(
                lutT_hbm.at[:, pl.ds((lo + i) * _CB, _CB)], tbufs[p], isems[p]
            ).wait()

            @pl.when(i >= 2)
            def _():
                pltpu.make_async_copy(
                    obufs[p].at[pl.ds(0, _CB * _DP)],
                    out_hbm.at[pl.ds((lo + i - 2) * (_CB * _DP), _CB * _DP)],
                    osems[p],
                ).wait()

            transpose(p, _CB)
            store(i, p, _CB)
        return c

    lax.fori_loop(0, _BPW // 2, pair_body, 0)

    pltpu.make_async_copy(
        lutT_hbm.at[:, pl.ds((lo + _BPW - 1) * _CB, _CB)], tbufs[0], isems[0]
    ).wait()
    pltpu.make_async_copy(
        obufs[0].at[pl.ds(0, _CB * _DP)],
        out_hbm.at[pl.ds((lo + _BPW - 3) * (_CB * _DP), _CB * _DP)],
        osems[0],
    ).wait()
    transpose(0, _CB)
    store(_BPW - 1, 0, _CB)

    for p, back in ((1, 2), (0, 1)):
        pltpu.make_async_copy(
            obufs[p].at[pl.ds(0, _CB * _DP)],
            out_hbm.at[pl.ds((lo + _BPW - back) * (_CB * _DP), _CB * _DP)],
            osems[p],
        ).wait()

    @pl.when(wid < _EXTRA)
    def _():
        load(_BPW, 0).wait()
        transpose(0, _CB)
        store(_BPW, 0, _CB).wait()

    @pl.when(wid == _NW - 1)
    def _():
        pltpu.sync_copy(tail_hbm, obufs[0].at[pl.ds(0, _TAIL * _DP)])
        pltpu.sync_copy(
            obufs[0].at[pl.ds(0, _TAIL * _DP)],
            out_hbm.at[pl.ds(_NMB * _CB * _DP, _TAIL * _DP)],
        )


@functools.partial(
    pl.kernel,
    mesh=_mesh(),
    out_type=jax.ShapeDtypeStruct((_S, 1024, 128), jnp.float32),
    scratch_types=[
        pltpu.VMEM((2, _UB), jnp.int32),
        pltpu.VMEM((2, _UB, _DP), jnp.float32),
        pltpu.VMEM((2, 4, _RPAD, 129), jnp.float32),
        [pltpu.SemaphoreType.DMA] * 2,
        [pltpu.SemaphoreType.DMA] * 2,
    ],
    compiler_params=pltpu.CompilerParams(use_tc_tiling_on_sc=False, needs_layout_passes=False),
)
def _gather(xT_hbm, lut_hbm, out_hbm, idxb, rows, sbuf, gsems, ssems):
    wid = lax.axis_index("s") * _NC + lax.axis_index("c")
    u0 = wid * _UPW

    lane = lax.iota(jnp.int32, 16)
    dt0 = lane // 8
    dt1 = dt0 + 2
    rvec = lane % 8
    zero = lane * 0

    def start_unit(i, p):
        u = u0 + i
        s = u // _NQ
        q = u % _NQ
        pltpu.sync_copy(xT_hbm.at[s, pl.ds(q * _UB, _UB)], idxb.at[p])
        return pltpu.async_copy(lut_hbm.at[idxb.at[p]], rows.at[p], gsems[p])

    def process_unit(i, p):
        u = u0 + i
        s = u // _NQ
        q = u % _NQ

        @plsc.parallel_loop(0, _UB, 1, unroll=4)
        def _(j):
            btl = j // 128
            c = j % 128
            rv = rvec + btl * 8
            cv = zero + c
            for h, dtv in ((0, dt0), (1, dt1)):
                val = rows[p, j, pl.ds(16 * h, 16)] * _SCALE
                plsc.store_scatter(sbuf.at[p], [dtv, rv, cv], val)

        handles = []
        for dt in range(4):
            handles.append(pltpu.async_copy(
                sbuf.at[p, dt, pl.ds(0, 32), pl.ds(0, 128)],
                out_hbm.at[s, pl.ds(dt * 256 + q * 32, 32), :],
                ssems[p],
            ))
        return handles

    g = {0: start_unit(0, 0)}
    st = {}
    for i in range(_UPW):
        p = i % 2
        if i + 1 < _UPW:
            g[i + 1] = start_unit(i + 1, 1 - p)
        g.pop(i).wait()
        if i - 2 in st:
            for h in st.pop(i - 2):
                h.wait()
        st[i] = process_unit(i, p)
    for i in sorted(st):
        for h in st.pop(i):
            h.wait()


def kernel(x, lut):
    lutT = jnp.transpose(lut)
    tail = jnp.pad(lut[_NMB * _CB:], ((0, 0), (0, _DP - _D))).reshape(_TAIL * _DP)
    lin = _detile(lutT, tail)
    lut2 = lin.reshape(_VOCAB, _DP)
    xT = jnp.transpose(x).astype(jnp.int32)
    s4 = _gather(xT, lut2)
    s4 = s4.reshape(_S, 4, _B // 128, 8, 128)
    return jnp.transpose(s4, (2, 4, 0, 1, 3)).reshape(_B, _S, _D)


# speedup vs baseline: 2.8534x; 2.8534x over previous
import functools
import math

import jax
import jax.numpy as jnp
import numpy as np
from jax import lax
from jax.experimental import pallas as pl
from jax.experimental.pallas import tpu as pltpu
from jax.experimental.pallas import tpu_sc as plsc

_VOCAB = 1000000
_D = 32
_SCALE = np.float32(math.sqrt(_D))
_NC = 2
_NS = 16
_NW = _NC * _NS

_NBLK = _VOCAB // 128
_TAIL = _VOCAB - _NBLK * 128
_BPW = _NBLK // _NW
_EXTRA = _NBLK - _BPW * _NW

_B = 4096
_S = 200
_UB = 512
_NQ = _B // _UB
_UPW = _S * _NQ // _NW
_RPAD = 40


def _mesh():
    return plsc.VectorSubcoreMesh(core_axis_name="c", subcore_axis_name="s")


@functools.partial(
    pl.kernel,
    mesh=_mesh(),
    out_type=jax.ShapeDtypeStruct((_VOCAB * _D,), jnp.float32),
    scratch_types=[
        pltpu.VMEM((_D, 128), jnp.float32),
        pltpu.VMEM((_D, 128), jnp.float32),
        pltpu.VMEM((128 * _D,), jnp.float32),
        pltpu.VMEM((128 * _D,), jnp.float32),
        [pltpu.SemaphoreType.DMA] * 2,
        [pltpu.SemaphoreType.DMA] * 2,
    ],
    compiler_params=pltpu.CompilerParams(use_tc_tiling_on_sc=True, needs_layout_passes=False),
)
def _detile(lutT_hbm, tail_hbm, out_hbm, tbuf0, tbuf1, obuf0, obuf1, isems, osems):
    tbufs = (tbuf0, tbuf1)
    obufs = (obuf0, obuf1)
    wid = lax.axis_index("s") * _NC + lax.axis_index("c")
    lo = wid * _BPW + jnp.minimum(wid, _EXTRA)

    lane = lax.iota(jnp.int32, 16)

    def load(i, p):
        return pltpu.async_copy(
            lutT_hbm.at[:, pl.ds((lo + i) * 128, 128)], tbufs[p], isems[p]
        )

    def transpose(p):
        @plsc.parallel_loop(0, _D, 1, unroll=4)
        def _(dd):
            for c0 in range(8):
                val = tbufs[p][dd, pl.ds(16 * c0, 16)]
                idx = (16 * c0 + lane) * _D + dd
                plsc.store_scatter(obufs[p], [idx], val)

    def store(i, p):
        return pltpu.async_copy(
            obufs[p],
            out_hbm.at[pl.ds((lo + i) * (128 * _D), 128 * _D)],
            osems[p],
        )

    load(0, 0)

    def pair_body(i2, c):
        for k in (0, 1):
            i = 2 * i2 + k
            p = k

            @pl.when(i + 1 < _BPW)
            def _():
                load(i + 1, 1 - p)

            pltpu.make_async_copy(
                lutT_hbm.at[:, pl.ds((lo + i) * 128, 128)], tbufs[p], isems[p]
            ).wait()

            @pl.when(i >= 2)
            def _():
                pltpu.make_async_copy(
                    obufs[p],
                    out_hbm.at[pl.ds((lo + i - 2) * (128 * _D), 128 * _D)],
                    osems[p],
                ).wait()

            transpose(p)
            store(i, p)
        return c

    lax.fori_loop(0, _BPW // 2, pair_body, 0)

    for p, back in ((0, 2), (1, 1)):
        pltpu.make_async_copy(
            obufs[p],
            out_hbm.at[pl.ds((lo + _BPW - back) * (128 * _D), 128 * _D)],
            osems[p],
        ).wait()

    @pl.when(wid < _EXTRA)
    def _():
        load(_BPW, 0).wait()
        transpose(0)
        store(_BPW, 0).wait()

    @pl.when(wid == _NW - 1)
    def _():
        pltpu.sync_copy(tail_hbm, obufs[0].at[pl.ds(0, _TAIL * _D)])
        pltpu.sync_copy(
            obufs[0].at[pl.ds(0, _TAIL * _D)],
            out_hbm.at[pl.ds(_NBLK * 128 * _D, _TAIL * _D)],
        )


@functools.partial(
    pl.kernel,
    mesh=_mesh(),
    out_type=jax.ShapeDtypeStruct((_S, 1024, 128), jnp.float32),
    scratch_types=[
        pltpu.VMEM((2, _UB), jnp.int32),
        pltpu.VMEM((2, _UB, _D), jnp.float32),
        pltpu.VMEM((2, 4, _RPAD, 129), jnp.float32),
        [pltpu.SemaphoreType.DMA] * 2,
        [pltpu.SemaphoreType.DMA] * 2,
    ],
    compiler_params=pltpu.CompilerParams(use_tc_tiling_on_sc=False, needs_layout_passes=False),
)
def _gather(xT_hbm, lut_hbm, out_hbm, idxb, rows, sbuf, gsems, ssems):
    wid = lax.axis_index("s") * _NC + lax.axis_index("c")
    u0 = wid * _UPW

    lane = lax.iota(jnp.int32, 16)
    dt0 = lane // 8
    dt1 = dt0 + 2
    rvec = lane % 8
    zero = lane * 0

    def start_unit(i, p):
        u = u0 + i
        s = u // _NQ
        q = u % _NQ
        pltpu.sync_copy(xT_hbm.at[s, pl.ds(q * _UB, _UB)], idxb.at[p])
        return pltpu.async_copy(lut_hbm.at[idxb.at[p]], rows.at[p], gsems[p])

    def process_unit(i, p):
        u = u0 + i
        s = u // _NQ
        q = u % _NQ

        @plsc.parallel_loop(0, _UB, 1, unroll=4)
        def _(j):
            btl = j // 128
            c = j % 128
            rv = rvec + btl * 8
            cv = zero + c
            for h, dtv in ((0, dt0), (1, dt1)):
                val = rows[p, j, pl.ds(16 * h, 16)] * _SCALE
                plsc.store_scatter(sbuf.at[p], [dtv, rv, cv], val)

        handles = []
        for dt in range(4):
            handles.append(pltpu.async_copy(
                sbuf.at[p, dt, pl.ds(0, 32), pl.ds(0, 128)],
                out_hbm.at[s, pl.ds(dt * 256 + q * 32, 32), :],
                ssems[p],
            ))
        return handles

    g = {0: start_unit(0, 0)}
    st = {}
    for i in range(_UPW):
        p = i % 2
        if i + 1 < _UPW:
            g[i + 1] = start_unit(i + 1, 1 - p)
        g.pop(i).wait()
        if i - 2 in st:
            for h in st.pop(i - 2):
                h.wait()
        st[i] = process_unit(i, p)
    for i in sorted(st):
        for h in st.pop(i):
            h.wait()


def kernel(x, lut):
    lutT = jnp.transpose(lut)
    tail = lut[_NBLK * 128:].reshape(_TAIL * _D)
    lin = _detile(lutT, tail)
    lut2 = lin.reshape(_VOCAB, _D)
    xT = jnp.transpose(x).astype(jnp.int32)
    s4 = _gather(xT, lut2)
    s4 = s4.reshape(_S, 4, _B // 128, 8, 128)
    return jnp.transpose(s4, (2, 4, 0, 1, 3)).reshape(_B, _S, _D)


# speedup vs baseline: 5.4123x; 1.8968x over previous
import functools
import math

import jax
import jax.numpy as jnp
import numpy as np
from jax import lax
from jax.experimental import pallas as pl
from jax.experimental.pallas import tpu as pltpu
from jax.experimental.pallas import tpu_sc as plsc

_VOCAB = 1000000
_D = 32
_SCALE = np.float32(math.sqrt(_D))
_NC = 2
_NS = 16
_NW = _NC * _NS

_NBLK = _VOCAB // 128
_TAIL = _VOCAB - _NBLK * 128
_BPW = _NBLK // _NW
_EXTRA = _NBLK - _BPW * _NW

_B = 4096
_S = 200
_UB = 512
_NQ = _B // _UB
_UPW = _S * _NQ // _NW
_RPAD = 40


def _mesh():
    return plsc.VectorSubcoreMesh(core_axis_name="c", subcore_axis_name="s")


@functools.partial(
    pl.kernel,
    mesh=_mesh(),
    out_type=jax.ShapeDtypeStruct((_VOCAB * _D,), jnp.float32),
    scratch_types=[
        pltpu.VMEM((_D, 128), jnp.float32),
        pltpu.VMEM((_D, 128), jnp.float32),
        pltpu.VMEM((128 * _D,), jnp.float32),
        pltpu.VMEM((128 * _D,), jnp.float32),
        [pltpu.SemaphoreType.DMA] * 2,
        [pltpu.SemaphoreType.DMA] * 2,
    ],
    compiler_params=pltpu.CompilerParams(use_tc_tiling_on_sc=True, needs_layout_passes=False),
)
def _detile(lutT_hbm, tail_hbm, out_hbm, tbuf0, tbuf1, obuf0, obuf1, isems, osems):
    tbufs = (tbuf0, tbuf1)
    obufs = (obuf0, obuf1)
    wid = lax.axis_index("s") * _NC + lax.axis_index("c")
    lo = wid * _BPW + jnp.minimum(wid, _EXTRA)

    lane = lax.iota(jnp.int32, 16)

    def load(i, p):
        return pltpu.async_copy(
            lutT_hbm.at[:, pl.ds((lo + i) * 128, 128)], tbufs[p], isems[p]
        )

    def transpose(p):
        @plsc.parallel_loop(0, _D, 1, unroll=4)
        def _(dd):
            rvec = (dd + lane) & (_D - 1)
            for c0 in range(8):
                cvec = 16 * c0 + lane
                val = plsc.load_gather(tbufs[p], [rvec, cvec])
                plsc.store_scatter(obufs[p], [cvec * _D + rvec], val)

    def store(i, p):
        return pltpu.async_copy(
            obufs[p],
            out_hbm.at[pl.ds((lo + i) * (128 * _D), 128 * _D)],
            osems[p],
        )

    load(0, 0)

    def pair_body(i2, c):
        for k in (0, 1):
            i = 2 * i2 + k
            p = k

            @pl.when(i + 1 < _BPW)
            def _():
                load(i + 1, 1 - p)

            pltpu.make_async_copy(
                lutT_hbm.at[:, pl.ds((lo + i) * 128, 128)], tbufs[p], isems[p]
            ).wait()

            @pl.when(i >= 2)
            def _():
                pltpu.make_async_copy(
                    obufs[p],
                    out_hbm.at[pl.ds((lo + i - 2) * (128 * _D), 128 * _D)],
                    osems[p],
                ).wait()

            transpose(p)
            store(i, p)
        return c

    lax.fori_loop(0, _BPW // 2, pair_body, 0)

    for p, back in ((0, 2), (1, 1)):
        pltpu.make_async_copy(
            obufs[p],
            out_hbm.at[pl.ds((lo + _BPW - back) * (128 * _D), 128 * _D)],
            osems[p],
        ).wait()

    @pl.when(wid < _EXTRA)
    def _():
        load(_BPW, 0).wait()
        transpose(0)
        store(_BPW, 0).wait()

    @pl.when(wid == _NW - 1)
    def _():
        pltpu.sync_copy(tail_hbm, obufs[0].at[pl.ds(0, _TAIL * _D)])
        pltpu.sync_copy(
            obufs[0].at[pl.ds(0, _TAIL * _D)],
            out_hbm.at[pl.ds(_NBLK * 128 * _D, _TAIL * _D)],
        )


@functools.partial(
    pl.kernel,
    mesh=_mesh(),
    out_type=jax.ShapeDtypeStruct((_S, 1024, 128), jnp.float32),
    scratch_types=[
        pltpu.VMEM((2, _UB), jnp.int32),
        pltpu.VMEM((2, _UB, _D), jnp.float32),
        pltpu.VMEM((2, 4, _RPAD, 129), jnp.float32),
        [pltpu.SemaphoreType.DMA] * 2,
        [pltpu.SemaphoreType.DMA] * 2,
    ],
    compiler_params=pltpu.CompilerParams(use_tc_tiling_on_sc=False, needs_layout_passes=False),
)
def _gather(xT_hbm, lut_hbm, out_hbm, idxb, rows, sbuf, gsems, ssems):
    wid = lax.axis_index("s") * _NC + lax.axis_index("c")
    u0 = wid * _UPW

    lane = lax.iota(jnp.int32, 16)
    dt0 = lane // 8
    dt1 = dt0 + 2
    rvec = lane % 8
    zero = lane * 0

    def start_unit(i, p):
        u = u0 + i
        s = u // _NQ
        q = u % _NQ
        pltpu.sync_copy(xT_hbm.at[s, pl.ds(q * _UB, _UB)], idxb.at[p])
        return pltpu.async_copy(lut_hbm.at[idxb.at[p]], rows.at[p], gsems[p])

    def process_unit(i, p):
        u = u0 + i
        s = u // _NQ
        q = u % _NQ

        @plsc.parallel_loop(0, _UB, 1, unroll=4)
        def _(j):
            btl = j // 128
            c = j % 128
            rv = rvec + btl * 8
            cv = zero + c
            for h, dtv in ((0, dt0), (1, dt1)):
                val = rows[p, j, pl.ds(16 * h, 16)] * _SCALE
                plsc.store_scatter(sbuf.at[p], [dtv, rv, cv], val)

        handles = []
        for dt in range(4):
            handles.append(pltpu.async_copy(
                sbuf.at[p, dt, pl.ds(0, 32), pl.ds(0, 128)],
                out_hbm.at[s, pl.ds(dt * 256 + q * 32, 32), :],
                ssems[p],
            ))
        return handles

    g = {0: start_unit(0, 0)}
    st = {}
    for i in range(_UPW):
        p = i % 2
        if i + 1 < _UPW:
            g[i + 1] = start_unit(i + 1, 1 - p)
        g.pop(i).wait()
        if i - 2 in st:
            for h in st.pop(i - 2):
                h.wait()
        st[i] = process_unit(i, p)
    for i in sorted(st):
        for h in st.pop(i):
            h.wait()


def kernel(x, lut):
    lutT = jnp.transpose(lut)
    tail = lut[_NBLK * 128:].reshape(_TAIL * _D)
    lin = _detile(lutT, tail)
    lut2 = lin.reshape(_VOCAB, _D)
    xT = jnp.transpose(x).astype(jnp.int32)
    s4 = _gather(xT, lut2)
    s4 = s4.reshape(_S, 4, _B // 128, 8, 128)
    return jnp.transpose(s4, (2, 4, 0, 1, 3)).reshape(_B, _S, _D)


# speedup vs baseline: 6.0193x; 1.1122x over previous
import functools
import math

import jax
import jax.numpy as jnp
import numpy as np
from jax import lax
from jax.experimental import pallas as pl
from jax.experimental.pallas import tpu as pltpu
from jax.experimental.pallas import tpu_sc as plsc

_VOCAB = 1000000
_D = 32
_SCALE = np.float32(math.sqrt(_D))
_NC = 2
_NS = 16
_NW = _NC * _NS

_NBLK = _VOCAB // 128
_TAIL = _VOCAB - _NBLK * 128
_BPW = _NBLK // _NW
_EXTRA = _NBLK - _BPW * _NW

_B = 4096
_S = 200
_UB = 512
_NQ = _B // _UB
_UPW = _S * _NQ // _NW
_RPAD = 40


def _mesh():
    return plsc.VectorSubcoreMesh(core_axis_name="c", subcore_axis_name="s")


@functools.partial(
    pl.kernel,
    mesh=_mesh(),
    out_type=jax.ShapeDtypeStruct((_VOCAB * _D,), jnp.float32),
    scratch_types=[
        [pltpu.VMEM((_D, 128), jnp.float32)] * 4,
        [pltpu.VMEM((128 * _D,), jnp.float32)] * 4,
        [pltpu.SemaphoreType.DMA] * 4,
        [pltpu.SemaphoreType.DMA] * 4,
    ],
    compiler_params=pltpu.CompilerParams(use_tc_tiling_on_sc=True, needs_layout_passes=False),
)
def _detile(lutT_hbm, tail_hbm, out_hbm, tbufs, obufs, isems, osems):
    wid = lax.axis_index("s") * _NC + lax.axis_index("c")
    lo = wid * _BPW + jnp.minimum(wid, _EXTRA)

    lane = lax.iota(jnp.int32, 16)

    def load(i, p):
        return pltpu.async_copy(
            lutT_hbm.at[:, pl.ds((lo + i) * 128, 128)], tbufs[p], isems[p]
        )

    def transpose(p):
        @plsc.parallel_loop(0, _D, 1, unroll=8)
        def _(dd):
            rvec = (dd + lane) & (_D - 1)
            for c0 in range(8):
                cvec = 16 * c0 + lane
                val = plsc.load_gather(tbufs[p], [rvec, cvec])
                plsc.store_scatter(obufs[p], [cvec * _D + rvec], val)

    def store(i, p):
        return pltpu.async_copy(
            obufs[p],
            out_hbm.at[pl.ds((lo + i) * (128 * _D), 128 * _D)],
            osems[p],
        )

    load(0, 0)
    load(1, 1)

    def quad_body(i4, c):
        for k in (0, 1, 2, 3):
            i = 4 * i4 + k
            p = k

            @pl.when(i + 2 < _BPW)
            def _():
                load(i + 2, (k + 2) % 4)

            pltpu.make_async_copy(
                lutT_hbm.at[:, pl.ds((lo + i) * 128, 128)], tbufs[p], isems[p]
            ).wait()

            @pl.when(i >= 4)
            def _():
                pltpu.make_async_copy(
                    obufs[p],
                    out_hbm.at[pl.ds((lo + i - 4) * (128 * _D), 128 * _D)],
                    osems[p],
                ).wait()

            transpose(p)
            store(i, p)
        return c

    lax.fori_loop(0, _BPW // 4, quad_body, 0)

    for back in (4, 3, 2, 1):
        p = (_BPW - back) % 4
        pltpu.make_async_copy(
            obufs[p],
            out_hbm.at[pl.ds((lo + _BPW - back) * (128 * _D), 128 * _D)],
            osems[p],
        ).wait()

    @pl.when(wid < _EXTRA)
    def _():
        load(_BPW, 0).wait()
        transpose(0)
        store(_BPW, 0).wait()

    @pl.when(wid == _NW - 1)
    def _():
        pltpu.sync_copy(tail_hbm, obufs[0].at[pl.ds(0, _TAIL * _D)])
        pltpu.sync_copy(
            obufs[0].at[pl.ds(0, _TAIL * _D)],
            out_hbm.at[pl.ds(_NBLK * 128 * _D, _TAIL * _D)],
        )


@functools.partial(
    pl.kernel,
    mesh=_mesh(),
    out_type=jax.ShapeDtypeStruct((_S, 1024, 128), jnp.float32),
    scratch_types=[
        pltpu.VMEM((3, _UB), jnp.int32),
        pltpu.VMEM((3, _UB, _D), jnp.float32),
        pltpu.VMEM((2, 4, _RPAD, 129), jnp.float32),
        [pltpu.SemaphoreType.DMA] * 3,
        [pltpu.SemaphoreType.DMA] * 2,
    ],
    compiler_params=pltpu.CompilerParams(use_tc_tiling_on_sc=False, needs_layout_passes=False),
)
def _gather(xT_hbm, lut_hbm, out_hbm, idxb, rows, sbuf, gsems, ssems):
    wid = lax.axis_index("s") * _NC + lax.axis_index("c")
    u0 = wid * _UPW

    lane = lax.iota(jnp.int32, 16)
    dt0 = lane // 8
    dt1 = dt0 + 2
    rvec = lane % 8
    zero = lane * 0

    def start_unit(i, p):
        u = u0 + i
        s = u // _NQ
        q = u % _NQ
        pltpu.sync_copy(xT_hbm.at[s, pl.ds(q * _UB, _UB)], idxb.at[p])
        return pltpu.async_copy(lut_hbm.at[idxb.at[p]], rows.at[p], gsems[p])

    def process_unit(i, pr, p):
        u = u0 + i
        s = u // _NQ
        q = u % _NQ

        @plsc.parallel_loop(0, _UB, 1, unroll=8)
        def _(j):
            btl = j // 128
            c = j % 128
            rv = rvec + btl * 8
            cv = zero + c
            for h, dtv in ((0, dt0), (1, dt1)):
                val = rows[pr, j, pl.ds(16 * h, 16)] * _SCALE
                plsc.store_scatter(sbuf.at[p], [dtv, rv, cv], val)

        handles = []
        for dt in range(4):
            handles.append(pltpu.async_copy(
                sbuf.at[p, dt, pl.ds(0, 32), pl.ds(0, 128)],
                out_hbm.at[s, pl.ds(dt * 256 + q * 32, 32), :],
                ssems[p],
            ))
        return handles

    g = {0: start_unit(0, 0), 1: start_unit(1, 1)}
    st = {}
    for i in range(_UPW):
        p = i % 2
        if i + 2 < _UPW:
            g[i + 2] = start_unit(i + 2, (i + 2) % 3)
        g.pop(i).wait()
        if i - 2 in st:
            for h in st.pop(i - 2):
                h.wait()
        st[i] = process_unit(i, i % 3, p)
    for i in sorted(st):
        for h in st.pop(i):
            h.wait()


def kernel(x, lut):
    lutT = jnp.transpose(lut)
    tail = lut[_NBLK * 128:].reshape(_TAIL * _D)
    lin = _detile(lutT, tail)
    lut2 = lin.reshape(_VOCAB, _D)
    xT = jnp.transpose(x).astype(jnp.int32)
    s4 = _gather(xT, lut2)
    s4 = s4.reshape(_S, 4, _B // 128, 8, 128)
    return jnp.transpose(s4, (2, 4, 0, 1, 3)).reshape(_B, _S, _D)


# speedup vs baseline: 6.4240x; 1.0672x over previous
import functools
import math

import jax
import jax.numpy as jnp
import numpy as np
from jax import lax
from jax.experimental import pallas as pl
from jax.experimental.pallas import tpu as pltpu
from jax.experimental.pallas import tpu_sc as plsc

_VOCAB = 1000000
_D = 32
_SCALE = np.float32(math.sqrt(_D))
_NC = 2
_NS = 16
_NW = _NC * _NS

_NBLK = _VOCAB // 128
_TAIL = _VOCAB - _NBLK * 128
_BPW = _NBLK // _NW
_EXTRA = _NBLK - _BPW * _NW

_B = 4096
_S = 200
_UB = 512
_NQ = _B // _UB
_UPW = _S * _NQ // _NW
_RPAD = 40


def _mesh():
    return plsc.VectorSubcoreMesh(core_axis_name="c", subcore_axis_name="s")


@functools.partial(
    pl.kernel,
    mesh=_mesh(),
    out_type=jax.ShapeDtypeStruct((_VOCAB * _D,), jnp.float32),
    scratch_types=[
        [pltpu.VMEM((_D, 128), jnp.float32)] * 4,
        [pltpu.VMEM((128 * _D,), jnp.float32)] * 4,
        [pltpu.SemaphoreType.DMA] * 4,
        [pltpu.SemaphoreType.DMA] * 4,
    ],
    compiler_params=pltpu.CompilerParams(use_tc_tiling_on_sc=True, needs_layout_passes=False),
)
def _detile(lutT_hbm, tail_hbm, out_hbm, tbufs, obufs, isems, osems):
    wid = lax.axis_index("s") * _NC + lax.axis_index("c")
    lo = wid * _BPW + jnp.minimum(wid, _EXTRA)

    lane = lax.iota(jnp.int32, 16)

    def load(i, p):
        return pltpu.async_copy(
            lutT_hbm.at[:, pl.ds((lo + i) * 128, 128)], tbufs[p], isems[p]
        )

    def transpose(p):
        @plsc.parallel_loop(0, _D, 1, unroll=8)
        def _(dd):
            rvec = (dd + lane) & (_D - 1)
            for c0 in range(8):
                cvec = 16 * c0 + lane
                val = plsc.load_gather(tbufs[p], [rvec, cvec])
                plsc.store_scatter(obufs[p], [cvec * _D + rvec], val)

    def store(i, p):
        return pltpu.async_copy(
            obufs[p],
            out_hbm.at[pl.ds((lo + i) * (128 * _D), 128 * _D)],
            osems[p],
        )

    load(0, 0)
    load(1, 1)

    def quad_body(i4, c):
        for k in (0, 1, 2, 3):
            i = 4 * i4 + k
            p = k

            @pl.when(i + 2 < _BPW)
            def _():
                load(i + 2, (k + 2) % 4)

            pltpu.make_async_copy(
                lutT_hbm.at[:, pl.ds((lo + i) * 128, 128)], tbufs[p], isems[p]
            ).wait()

            @pl.when(i >= 4)
            def _():
                pltpu.make_async_copy(
                    obufs[p],
                    out_hbm.at[pl.ds((lo + i - 4) * (128 * _D), 128 * _D)],
                    osems[p],
                ).wait()

            transpose(p)
            store(i, p)
        return c

    lax.fori_loop(0, _BPW // 4, quad_body, 0)

    for back in (4, 3, 2, 1):
        p = (_BPW - back) % 4
        pltpu.make_async_copy(
            obufs[p],
            out_hbm.at[pl.ds((lo + _BPW - back) * (128 * _D), 128 * _D)],
            osems[p],
        ).wait()

    @pl.when(wid < _EXTRA)
    def _():
        load(_BPW, 0).wait()
        transpose(0)
        store(_BPW, 0).wait()

    @pl.when(wid == _NW - 1)
    def _():
        pltpu.sync_copy(tail_hbm, obufs[0].at[pl.ds(0, _TAIL * _D)])
        pltpu.sync_copy(
            obufs[0].at[pl.ds(0, _TAIL * _D)],
            out_hbm.at[pl.ds(_NBLK * 128 * _D, _TAIL * _D)],
        )


@functools.partial(
    pl.kernel,
    mesh=_mesh(),
    out_type=jax.ShapeDtypeStruct((_S, _B * _D), jnp.float32),
    scratch_types=[
        pltpu.VMEM((3, _UB), jnp.int32),
        pltpu.VMEM((3, _UB, _D), jnp.float32),
        pltpu.VMEM((2, _UB * _D), jnp.float32),
        [pltpu.SemaphoreType.DMA] * 3,
        [pltpu.SemaphoreType.DMA] * 2,
    ],
    compiler_params=pltpu.CompilerParams(use_tc_tiling_on_sc=False, needs_layout_passes=False),
)
def _gather(xT_hbm, lut_hbm, out_hbm, idxb, rows, sbuf, gsems, ssems):
    wid = lax.axis_index("s") * _NC + lax.axis_index("c")
    u0 = wid * _UPW

    lane = lax.iota(jnp.int32, 16)

    def start_unit(i, p):
        u = u0 + i
        s = u // _NQ
        q = u % _NQ
        pltpu.sync_copy(xT_hbm.at[s, pl.ds(q * _UB, _UB)], idxb.at[p])
        return pltpu.async_copy(lut_hbm.at[idxb.at[p]], rows.at[p], gsems[p])

    def process_unit(i, pr, p):
        u = u0 + i
        s = u // _NQ
        q = u % _NQ

        @plsc.parallel_loop(0, 1024, 1, unroll=8)
        def _(t):
            d0 = t >> 5
            j0 = (t & 31) * 16
            dvec = (d0 + lane) & (_D - 1)
            dconst = (dvec >> 3) * 4096 + (dvec & 7) * 128 + lane
            sidx = dconst + ((j0 >> 7) * 1024 + (j0 & 127))
            val = plsc.load_gather(rows.at[pr], [j0 + lane, dvec])
            plsc.store_scatter(sbuf.at[p], [sidx], val * _SCALE)

        handles = []
        for dt in range(4):
            handles.append(pltpu.async_copy(
                sbuf.at[p, pl.ds(dt * 4096, 4096)],
                out_hbm.at[s, pl.ds(dt * 32768 + q * 4096, 4096)],
                ssems[p],
            ))
        return handles

    g = {0: start_unit(0, 0), 1: start_unit(1, 1)}
    st = {}
    for i in range(_UPW):
        p = i % 2
        if i + 2 < _UPW:
            g[i + 2] = start_unit(i + 2, (i + 2) % 3)
        g.pop(i).wait()
        if i - 2 in st:
            for h in st.pop(i - 2):
                h.wait()
        st[i] = process_unit(i, i % 3, p)
    for i in sorted(st):
        for h in st.pop(i):
            h.wait()


def kernel(x, lut):
    lutT = jnp.transpose(lut)
    tail = lut[_NBLK * 128:].reshape(_TAIL * _D)
    lin = _detile(lutT, tail)
    lut2 = lin.reshape(_VOCAB, _D)
    xT = jnp.transpose(x).astype(jnp.int32)
    s4 = _gather(xT, lut2)
    s4 = s4.reshape(_S, 4, _B // 128, 8, 128)
    return jnp.transpose(s4, (2, 4, 0, 1, 3)).reshape(_B, _S, _D)
